# baseline (speedup 1.0000x reference)
import functools

import jax
import jax.numpy as jnp
from jax.experimental import pallas as pl
from jax.experimental.pallas import tpu as pltpu

QUES = 2048
HID = 1024
T = 32768
K = 32

_INTERPRET = False

_GA = 32
_BT = T // _GA
_BH = HID // _GA


def _stage_a_body(qblk_ref, qrow_ref, s_ref, h0r_ref, h0c_ref, wih_ref,
                  whh_ref, bih_ref, bhh_ref, wsc_ref, bsc_ref,
                  alpha_ref, h_ref, predq_ref):
    j = pl.program_id(0)
    q = qrow_ref[...]
    av = jax.lax.dot_general(
        q, qblk_ref[...], (((1,), (1,)), ((), ())),
        preferred_element_type=jnp.float32)
    alpha_ref[...] = av[0]

    s = s_ref[0, 0]
    m_ge = (s >= 0.5).astype(jnp.float32)
    m_lt = (s < 0.5).astype(jnp.float32)
    x = jnp.concatenate([q * m_ge, q * m_lt], axis=1)

    gi3 = jax.lax.dot_general(
        wih_ref[...], x, (((2,), (1,)), ((), ())),
        preferred_element_type=jnp.float32)
    gh3 = jax.lax.dot_general(
        whh_ref[...], h0r_ref[...], (((2,), (1,)), ((), ())),
        preferred_element_type=jnp.float32)
    i_r, i_z, i_n = gi3[0] + bih_ref[0], gi3[1] + bih_ref[1], gi3[2] + bih_ref[2]
    h_r, h_z, h_n = gh3[0] + bhh_ref[0], gh3[1] + bhh_ref[1], gh3[2] + bhh_ref[2]
    r = jax.nn.sigmoid(i_r + h_r)
    z = jax.nn.sigmoid(i_z + h_z)
    n = jnp.tanh(i_n + r * h_n)
    h_ref[...] = (1.0 - z) * n + z * h0c_ref[...]

    @pl.when(j == 0)
    def _():
        predq_ref[0, 0] = jnp.sum(q * wsc_ref[:, :QUES]) + bsc_ref[0, 0]


def _stage_a(questions, question, score, h0, W_score, b_score,
             W_ih, W_hh, b_ih, b_hh):
    wih3 = W_ih.reshape(3, HID, 2 * QUES)
    whh3 = W_hh.reshape(3, HID, HID)
    bih3 = b_ih.reshape(3, HID, 1)
    bhh3 = b_hh.reshape(3, HID, 1)
    return pl.pallas_call(
        _stage_a_body,
        grid=(_GA,),
        in_specs=[
            pl.BlockSpec((_BT, QUES), lambda j: (j, 0)),
            pl.BlockSpec((1, QUES), lambda j: (0, 0)),
            pl.BlockSpec((1, 1), lambda j: (0, 0), memory_space=pltpu.SMEM),
            pl.BlockSpec((1, HID), lambda j: (0, 0)),
            pl.BlockSpec((_BH, 1), lambda j: (j, 0)),
            pl.BlockSpec((3, _BH, 2 * QUES), lambda j: (0, j, 0)),
            pl.BlockSpec((3, _BH, HID), lambda j: (0, j, 0)),
            pl.BlockSpec((3, _BH, 1), lambda j: (0, j, 0)),
            pl.BlockSpec((3, _BH, 1), lambda j: (0, j, 0)),
            pl.BlockSpec((1, QUES + HID), lambda j: (0, 0)),
            pl.BlockSpec((1, 1), lambda j: (0, 0), memory_space=pltpu.SMEM),
        ],
        out_specs=[
            pl.BlockSpec((_BT,), lambda j: (j,)),
            pl.BlockSpec((_BH, 1), lambda j: (j, 0)),
            pl.BlockSpec((1, 1), lambda j: (0, 0), memory_space=pltpu.SMEM),
        ],
        out_shape=[
            jax.ShapeDtypeStruct((T,), jnp.float32),
            jax.ShapeDtypeStruct((HID, 1), jnp.float32),
            jax.ShapeDtypeStruct((1, 1), jnp.float32),
        ],
        interpret=_INTERPRET,
    )(questions, question.reshape(1, QUES), score.reshape(1, 1),
      h0.reshape(1, HID), h0.reshape(HID, 1), wih3, whh3, bih3, bhh3,
      W_score, b_score.reshape(1, 1))


def _attn_body(alpha_ref, hs_ref, wsca_ref, predq_ref, pred_ref,
               rows_ref, sems):
    a = alpha_ref[...]
    flat = jax.lax.broadcasted_iota(jnp.int32, a.shape, 0)
    neg = jnp.float32(-jnp.inf)
    big = jnp.int32(2**30)
    vals = []
    for t in range(K):
        m = jnp.max(a)
        eq = a == m
        fi = jnp.min(jnp.where(eq, flat, big))
        pltpu.make_async_copy(hs_ref.at[fi], rows_ref.at[t], sems.at[t]).start()
        vals.append(m)
        a = jnp.where(flat == fi, neg, a)
    m0 = vals[0]
    ws = [jnp.exp(v - m0) for v in vals]
    z = functools.reduce(lambda x, y: x + y, ws)
    acc = jnp.zeros((1, HID), dtype=jnp.float32)
    for t in range(K):
        pltpu.make_async_copy(hs_ref.at[0], rows_ref.at[t], sems.at[t]).wait()
        acc = acc + rows_ref[t] * (ws[t] / z)
    pred_ref[0, 0] = predq_ref[0, 0] + jnp.sum(acc * wsca_ref[...])


def _attn(alpha, hs, wsc_attn, predq):
    return pl.pallas_call(
        _attn_body,
        in_specs=[
            pl.BlockSpec(memory_space=pltpu.VMEM),
            pl.BlockSpec(memory_space=pl.ANY),
            pl.BlockSpec(memory_space=pltpu.VMEM),
            pl.BlockSpec(memory_space=pltpu.SMEM),
        ],
        out_specs=pl.BlockSpec(memory_space=pltpu.SMEM),
        out_shape=jax.ShapeDtypeStruct((1, 1), jnp.float32),
        scratch_shapes=[
            pltpu.VMEM((K, 1, HID), jnp.float32),
            pltpu.SemaphoreType.DMA((K,)),
        ],
        interpret=_INTERPRET,
    )(alpha, hs, wsc_attn, predq)


def kernel(question, score, questions, hs, W_score, b_score, W_ih, W_hh, b_ih, b_hh):
    h0 = hs[T - 1, 0]
    alpha, h_new, predq = _stage_a(questions, question, score, h0, W_score,
                                   b_score, W_ih, W_hh, b_ih, b_hh)
    pred = _attn(alpha, hs, W_score[:, QUES:], predq)
    return pred, h_new.reshape(1, 1, HID)


# speedup vs baseline: 1.0941x; 1.0941x over previous
import functools

import jax
import jax.numpy as jnp
from jax import lax
from jax.experimental import pallas as pl
from jax.experimental.pallas import tpu as pltpu
from jax.experimental.pallas import tpu_sc as plsc

QUES = 2048
HID = 1024
T = 32768
K = 32

_INTERPRET = False

_GA = 32
_BT = T // _GA
_BH = HID // _GA


def _stage_a_body(qblk_ref, qrow_ref, s_ref, h0r_ref, h0c_ref, wih_ref,
                  whh_ref, bih_ref, bhh_ref, wsc_ref, bsc_ref,
                  alpha_ref, h_ref, predq_ref):
    j = pl.program_id(0)
    q = qrow_ref[...]
    av = jax.lax.dot_general(
        q, qblk_ref[...], (((1,), (1,)), ((), ())),
        preferred_element_type=jnp.float32)
    alpha_ref[...] = av[0]

    s = s_ref[0, 0]
    m_ge = (s >= 0.5).astype(jnp.float32)
    m_lt = (s < 0.5).astype(jnp.float32)
    x = jnp.concatenate([q * m_ge, q * m_lt], axis=1)

    gi3 = jax.lax.dot_general(
        wih_ref[...], x, (((2,), (1,)), ((), ())),
        preferred_element_type=jnp.float32)
    gh3 = jax.lax.dot_general(
        whh_ref[...], h0r_ref[...], (((2,), (1,)), ((), ())),
        preferred_element_type=jnp.float32)
    i_r, i_z, i_n = gi3[0] + bih_ref[0], gi3[1] + bih_ref[1], gi3[2] + bih_ref[2]
    h_r, h_z, h_n = gh3[0] + bhh_ref[0], gh3[1] + bhh_ref[1], gh3[2] + bhh_ref[2]
    r = jax.nn.sigmoid(i_r + h_r)
    z = jax.nn.sigmoid(i_z + h_z)
    n = jnp.tanh(i_n + r * h_n)
    h_ref[...] = (1.0 - z) * n + z * h0c_ref[...]

    @pl.when(j == 0)
    def _():
        predq_ref[0, 0] = jnp.sum(q * wsc_ref[:, :QUES]) + bsc_ref[0, 0]


def _stage_a(questions, question, score, h0, W_score, b_score,
             W_ih, W_hh, b_ih, b_hh):
    wih3 = W_ih.reshape(3, HID, 2 * QUES)
    whh3 = W_hh.reshape(3, HID, HID)
    bih3 = b_ih.reshape(3, HID, 1)
    bhh3 = b_hh.reshape(3, HID, 1)
    return pl.pallas_call(
        _stage_a_body,
        grid=(_GA,),
        in_specs=[
            pl.BlockSpec((_BT, QUES), lambda j: (j, 0)),
            pl.BlockSpec((1, QUES), lambda j: (0, 0)),
            pl.BlockSpec((1, 1), lambda j: (0, 0), memory_space=pltpu.SMEM),
            pl.BlockSpec((1, HID), lambda j: (0, 0)),
            pl.BlockSpec((_BH, 1), lambda j: (j, 0)),
            pl.BlockSpec((3, _BH, 2 * QUES), lambda j: (0, j, 0)),
            pl.BlockSpec((3, _BH, HID), lambda j: (0, j, 0)),
            pl.BlockSpec((3, _BH, 1), lambda j: (0, j, 0)),
            pl.BlockSpec((3, _BH, 1), lambda j: (0, j, 0)),
            pl.BlockSpec((1, QUES + HID), lambda j: (0, 0)),
            pl.BlockSpec((1, 1), lambda j: (0, 0), memory_space=pltpu.SMEM),
        ],
        out_specs=[
            pl.BlockSpec((_BT,), lambda j: (j,)),
            pl.BlockSpec((_BH, 1), lambda j: (j, 0)),
            pl.BlockSpec((1, 1), lambda j: (0, 0), memory_space=pltpu.SMEM),
        ],
        out_shape=[
            jax.ShapeDtypeStruct((T,), jnp.float32),
            jax.ShapeDtypeStruct((HID, 1), jnp.float32),
            jax.ShapeDtypeStruct((1, 1), jnp.float32),
        ],
        interpret=_INTERPRET,
    )(questions, question.reshape(1, QUES), score.reshape(1, 1),
      h0.reshape(1, HID), h0.reshape(HID, 1), wih3, whh3, bih3, bhh3,
      W_score, b_score.reshape(1, 1))


def _attn_body(alpha_ref, hs_ref, wsca_ref, predq_ref, pred_ref,
               rows_ref, sems):
    a = alpha_ref[...]
    flat = jax.lax.broadcasted_iota(jnp.int32, a.shape, 0)
    neg = jnp.float32(-jnp.inf)
    big = jnp.int32(2**30)
    vals = []
    for t in range(K):
        m = jnp.max(a)
        eq = a == m
        fi = jnp.min(jnp.where(eq, flat, big))
        pltpu.make_async_copy(hs_ref.at[fi], rows_ref.at[t], sems.at[t]).start()
        vals.append(m)
        a = jnp.where(flat == fi, neg, a)
    m0 = vals[0]
    ws = [jnp.exp(v - m0) for v in vals]
    z = functools.reduce(lambda x, y: x + y, ws)
    acc = jnp.zeros((1, HID), dtype=jnp.float32)
    for t in range(K):
        pltpu.make_async_copy(hs_ref.at[0], rows_ref.at[t], sems.at[t]).wait()
        acc = acc + rows_ref[t] * (ws[t] / z)
    pred_ref[0, 0] = predq_ref[0, 0] + jnp.sum(acc * wsca_ref[...])


def _attn(alpha, hs, wsc_attn, predq):
    return pl.pallas_call(
        _attn_body,
        in_specs=[
            pl.BlockSpec(memory_space=pltpu.VMEM),
            pl.BlockSpec(memory_space=pl.ANY),
            pl.BlockSpec(memory_space=pltpu.VMEM),
            pl.BlockSpec(memory_space=pltpu.SMEM),
        ],
        out_specs=pl.BlockSpec(memory_space=pltpu.SMEM),
        out_shape=jax.ShapeDtypeStruct((1, 1), jnp.float32),
        scratch_shapes=[
            pltpu.VMEM((K, 1, HID), jnp.float32),
            pltpu.SemaphoreType.DMA((K,)),
        ],
        interpret=_INTERPRET,
    )(alpha, hs, wsc_attn, predq)


_NS = 16
_PER = T // _NS
_CH = _PER // 16


def _splat_lane(ref, x, lane):
    ref[...] = x
    return plsc.load_gather(ref, [jnp.zeros((16,), jnp.int32) + lane])


def _bfly_sum(ref, x):
    iota = lax.iota(jnp.int32, 16)
    v = x
    for sh in (8, 4, 2, 1):
        ref[...] = v
        v = v + plsc.load_gather(ref, [jnp.bitwise_xor(iota, sh)])
    return v


def _merge16(ak, av_, bk, bv):
    rbk = lax.rev(bk, (0,))
    rbv = lax.rev(bv, (0,))
    m = ak >= rbk
    hk = jnp.where(m, ak, rbk)
    hv = jnp.where(m, av_, rbv)
    lk = jnp.where(m, rbk, ak)
    lv = jnp.where(m, rbv, av_)
    lk, lv = plsc.sort_key_val(lk, lv)
    hk, hv = plsc.sort_key_val(hk, hv)
    return lk, lv, hk, hv


def _top32_update(carry, ck, cv):
    tlk, tlv, thk, thv = carry
    rbk = lax.rev(tlk, (0,))
    rbv = lax.rev(tlv, (0,))
    m = ck >= rbk
    uk = jnp.where(m, ck, rbk)
    uv = jnp.where(m, cv, rbv)
    uk, uv = plsc.sort_key_val(uk, uv)
    nlk, nlv, nhk, nhv = _merge16(thk, thv, uk, uv)
    return nlk, nlv, nhk, nhv


def _attn_sc_body(alpha_hbm, hs_hbm, wsca_hbm, predq_hbm, out_hbm,
                  av_ref, ckf_ref, cvf_ref, st32k_ref, st32v_ref,
                  i32_ref, w32_ref, idx2_ref, rows_ref, wsca_ref,
                  pq_ref, st16_ref, p256_ref,
                  shk_ref, shv_ref, shw_ref, shi_ref, shp_ref, sem):
    wid = lax.axis_index("s")
    iota = lax.iota(jnp.int32, 16)
    ninf = jnp.float32(-jnp.inf)
    init = (jnp.full((16,), ninf), jnp.zeros((16,), jnp.int32),
            jnp.full((16,), ninf), jnp.zeros((16,), jnp.int32))

    pltpu.sync_copy(alpha_hbm.at[pl.ds(wid * _PER, _PER)], av_ref)
    pltpu.sync_copy(wsca_hbm, wsca_ref)

    base = wid * _PER

    def step(i, carry):
        c = av_ref[pl.ds(i * 16, 16)]
        ck, cv = plsc.sort_key_val(c, iota + (base + i * 16))
        return _top32_update(carry, ck, cv)

    tlk, tlv, thk, thv = lax.fori_loop(0, _CH, step, init)

    st32k_ref[pl.ds(0, 16)] = tlk
    st32k_ref[pl.ds(16, 16)] = thk
    st32v_ref[pl.ds(0, 16)] = tlv
    st32v_ref[pl.ds(16, 16)] = thv
    pltpu.sync_copy(st32k_ref, shk_ref.at[pl.ds(wid * 32, 32)])
    pltpu.sync_copy(st32v_ref, shv_ref.at[pl.ds(wid * 32, 32)])
    plsc.subcore_barrier()

    @pl.when(wid == 0)
    def _():
        pltpu.sync_copy(shk_ref, ckf_ref)
        pltpu.sync_copy(shv_ref, cvf_ref)

        def step2(i, carry):
            ck = ckf_ref[pl.ds(i * 16, 16)]
            cv = cvf_ref[pl.ds(i * 16, 16)]
            ck, cv = plsc.sort_key_val(ck, cv)
            return _top32_update(carry, ck, cv)

        tlk0, tlv0, thk0, thv0 = lax.fori_loop(0, 2 * _NS, step2, init)
        m0 = _splat_lane(st16_ref, thk0, 15)
        e_lo = jnp.exp(tlk0 - m0)
        e_hi = jnp.exp(thk0 - m0)
        zvec = _bfly_sum(st16_ref, e_lo + e_hi)
        w32_ref[pl.ds(0, 16)] = e_lo / zvec
        w32_ref[pl.ds(16, 16)] = e_hi / zvec
        i32_ref[pl.ds(0, 16)] = tlv0
        i32_ref[pl.ds(16, 16)] = thv0
        pltpu.sync_copy(w32_ref, shw_ref)
        pltpu.sync_copy(i32_ref, shi_ref)
    plsc.subcore_barrier()

    pltpu.sync_copy(shw_ref, w32_ref)
    pltpu.sync_copy(shi_ref, i32_ref)
    gidx = plsc.load_gather(i32_ref, [jnp.where(iota < 2, iota + 2 * wid, 0)])
    plsc.store_scatter(idx2_ref, [iota], gidx, mask=iota < 2)
    pltpu.async_copy(hs_hbm.at[idx2_ref], rows_ref, sem).wait()
    part = jnp.zeros((16,), jnp.float32)
    for r in range(2):
        acc = jnp.zeros((16,), jnp.float32)
        for c in range(HID // 16):
            acc = acc + rows_ref[r, 0, pl.ds(c * 16, 16)] * wsca_ref[pl.ds(c * 16, 16)]
        gw = plsc.load_gather(w32_ref, [jnp.zeros((16,), jnp.int32) + (2 * wid + r)])
        part = part + acc * gw
    st16_ref[...] = _bfly_sum(pq_ref, part)
    pltpu.sync_copy(st16_ref, shp_ref.at[pl.ds(wid * 16, 16)])
    plsc.subcore_barrier()

    @pl.when(wid == 0)
    def _():
        pltpu.sync_copy(shp_ref, p256_ref)
        pltpu.sync_copy(predq_hbm, pq_ref)
        tot = jnp.zeros((16,), jnp.float32)
        for r in range(_NS):
            tot = tot + p256_ref[pl.ds(r * 16, 16)]
        st16_ref[...] = tot + pq_ref[...]
        pltpu.sync_copy(st16_ref, out_hbm)


def _attn_sc(alpha, hs, wsca, predq16):
    mesh = plsc.VectorSubcoreMesh(core_axis_name="c", subcore_axis_name="s",
                                  num_cores=1)
    f32 = jnp.float32
    i32 = jnp.int32
    run = pl.kernel(
        _attn_sc_body,
        out_type=jax.ShapeDtypeStruct((16,), f32),
        mesh=mesh,
        compiler_params=pltpu.CompilerParams(needs_layout_passes=False),
        scratch_types=[
            pltpu.VMEM((_PER,), f32),
            pltpu.VMEM((32 * _NS,), f32),
            pltpu.VMEM((32 * _NS,), i32),
            pltpu.VMEM((32,), f32),
            pltpu.VMEM((32,), i32),
            pltpu.VMEM((32,), i32),
            pltpu.VMEM((32,), f32),
            pltpu.VMEM((2,), i32),
            pltpu.VMEM((2, 1, HID), f32),
            pltpu.VMEM((HID,), f32),
            pltpu.VMEM((16,), f32),
            pltpu.VMEM((16,), f32),
            pltpu.VMEM((16 * _NS,), f32),
            pltpu.VMEM_SHARED((32 * _NS,), f32),
            pltpu.VMEM_SHARED((32 * _NS,), i32),
            pltpu.VMEM_SHARED((32,), f32),
            pltpu.VMEM_SHARED((32,), i32),
            pltpu.VMEM_SHARED((16 * _NS,), f32),
            pltpu.SemaphoreType.DMA,
        ],
    )
    return run(alpha, hs, wsca, predq16)


def kernel(question, score, questions, hs, W_score, b_score, W_ih, W_hh, b_ih, b_hh):
    h0 = hs[T - 1, 0]
    alpha, h_new, predq = _stage_a(questions, question, score, h0, W_score,
                                   b_score, W_ih, W_hh, b_ih, b_hh)
    predq16 = jnp.broadcast_to(predq.reshape(1), (16,))
    pred16 = _attn_sc(alpha, hs, W_score[0, QUES:], predq16)
    pred = pred16[:1].reshape(1, 1)
    return pred, h_new.reshape(1, 1, HID)


# speedup vs baseline: 1.0956x; 1.0014x over previous
import functools

import jax
import jax.numpy as jnp
from jax import lax
from jax.experimental import pallas as pl
from jax.experimental.pallas import tpu as pltpu
from jax.experimental.pallas import tpu_sc as plsc

QUES = 2048
HID = 1024
T = 32768
K = 32

_INTERPRET = False

_GA = 32
_BT = T // _GA
_BH = HID // _GA


def _stage_a_body(sel_ref, qblk_ref, qrow_ref, h0r_ref, h0c_ref, wih_ref,
                  whh_ref, bih_ref, bhh_ref, wsc_ref, bsc_ref,
                  alpha_ref, h_ref, predq_ref):
    j = pl.program_id(0)
    q = qrow_ref[...]
    av = jax.lax.dot_general(
        q, qblk_ref[...], (((1,), (1,)), ((), ())),
        preferred_element_type=jnp.float32)
    alpha_ref[...] = av[0]

    gi3 = jax.lax.dot_general(
        wih_ref[...], q, (((2,), (1,)), ((), ())),
        preferred_element_type=jnp.float32)
    gh3 = jax.lax.dot_general(
        whh_ref[...], h0r_ref[...], (((2,), (1,)), ((), ())),
        preferred_element_type=jnp.float32)
    i_r, i_z, i_n = gi3[0] + bih_ref[0], gi3[1] + bih_ref[1], gi3[2] + bih_ref[2]
    h_r, h_z, h_n = gh3[0] + bhh_ref[0], gh3[1] + bhh_ref[1], gh3[2] + bhh_ref[2]
    r = jax.nn.sigmoid(i_r + h_r)
    z = jax.nn.sigmoid(i_z + h_z)
    n = jnp.tanh(i_n + r * h_n)
    h_ref[...] = (1.0 - z) * n + z * h0c_ref[...]

    @pl.when(j == 0)
    def _():
        predq_ref[0, 0] = jnp.sum(q * wsc_ref[:, :QUES]) + bsc_ref[0, 0]


def _stage_a(questions, question, score, h0, W_score, b_score,
             W_ih, W_hh, b_ih, b_hh):
    wih3 = W_ih.reshape(3, HID, 2 * QUES)
    whh3 = W_hh.reshape(3, HID, HID)
    bih3 = b_ih.reshape(3, HID, 1)
    bhh3 = b_hh.reshape(3, HID, 1)
    sel = (score[0] < 0.5).astype(jnp.int32).reshape(1)
    grid_spec = pltpu.PrefetchScalarGridSpec(
        num_scalar_prefetch=1,
        grid=(_GA,),
        in_specs=[
            pl.BlockSpec((_BT, QUES), lambda j, sel: (j, 0)),
            pl.BlockSpec((1, QUES), lambda j, sel: (0, 0)),
            pl.BlockSpec((1, HID), lambda j, sel: (0, 0)),
            pl.BlockSpec((_BH, 1), lambda j, sel: (j, 0)),
            pl.BlockSpec((3, _BH, QUES), lambda j, sel: (0, j, sel[0])),
            pl.BlockSpec((3, _BH, HID), lambda j, sel: (0, j, 0)),
            pl.BlockSpec((3, _BH, 1), lambda j, sel: (0, j, 0)),
            pl.BlockSpec((3, _BH, 1), lambda j, sel: (0, j, 0)),
            pl.BlockSpec((1, QUES + HID), lambda j, sel: (0, 0)),
            pl.BlockSpec((1, 1), lambda j, sel: (0, 0),
                         memory_space=pltpu.SMEM),
        ],
        out_specs=[
            pl.BlockSpec((_BT,), lambda j, sel: (j,)),
            pl.BlockSpec((_BH, 1), lambda j, sel: (j, 0)),
            pl.BlockSpec((1, 1), lambda j, sel: (0, 0),
                         memory_space=pltpu.SMEM),
        ],
    )
    return pl.pallas_call(
        _stage_a_body,
        grid_spec=grid_spec,
        out_shape=[
            jax.ShapeDtypeStruct((T,), jnp.float32),
            jax.ShapeDtypeStruct((HID, 1), jnp.float32),
            jax.ShapeDtypeStruct((1, 1), jnp.float32),
        ],
        interpret=_INTERPRET,
    )(sel, questions, question.reshape(1, QUES),
      h0.reshape(1, HID), h0.reshape(HID, 1), wih3, whh3, bih3, bhh3,
      W_score, b_score.reshape(1, 1))


def _attn_body(alpha_ref, hs_ref, wsca_ref, predq_ref, pred_ref,
               rows_ref, sems):
    a = alpha_ref[...]
    flat = jax.lax.broadcasted_iota(jnp.int32, a.shape, 0)
    neg = jnp.float32(-jnp.inf)
    big = jnp.int32(2**30)
    vals = []
    for t in range(K):
        m = jnp.max(a)
        eq = a == m
        fi = jnp.min(jnp.where(eq, flat, big))
        pltpu.make_async_copy(hs_ref.at[fi], rows_ref.at[t], sems.at[t]).start()
        vals.append(m)
        a = jnp.where(flat == fi, neg, a)
    m0 = vals[0]
    ws = [jnp.exp(v - m0) for v in vals]
    z = functools.reduce(lambda x, y: x + y, ws)
    acc = jnp.zeros((1, HID), dtype=jnp.float32)
    for t in range(K):
        pltpu.make_async_copy(hs_ref.at[0], rows_ref.at[t], sems.at[t]).wait()
        acc = acc + rows_ref[t] * (ws[t] / z)
    pred_ref[0, 0] = predq_ref[0, 0] + jnp.sum(acc * wsca_ref[...])


def _attn(alpha, hs, wsc_attn, predq):
    return pl.pallas_call(
        _attn_body,
        in_specs=[
            pl.BlockSpec(memory_space=pltpu.VMEM),
            pl.BlockSpec(memory_space=pl.ANY),
            pl.BlockSpec(memory_space=pltpu.VMEM),
            pl.BlockSpec(memory_space=pltpu.SMEM),
        ],
        out_specs=pl.BlockSpec(memory_space=pltpu.SMEM),
        out_shape=jax.ShapeDtypeStruct((1, 1), jnp.float32),
        scratch_shapes=[
            pltpu.VMEM((K, 1, HID), jnp.float32),
            pltpu.SemaphoreType.DMA((K,)),
        ],
        interpret=_INTERPRET,
    )(alpha, hs, wsc_attn, predq)


_NS = 16
_PER = T // _NS
_CH = _PER // 16


def _splat_lane(ref, x, lane):
    ref[...] = x
    return plsc.load_gather(ref, [jnp.zeros((16,), jnp.int32) + lane])


def _bfly_sum(ref, x):
    iota = lax.iota(jnp.int32, 16)
    v = x
    for sh in (8, 4, 2, 1):
        ref[...] = v
        v = v + plsc.load_gather(ref, [jnp.bitwise_xor(iota, sh)])
    return v


def _merge16(ak, av_, bk, bv):
    rbk = lax.rev(bk, (0,))
    rbv = lax.rev(bv, (0,))
    m = ak >= rbk
    hk = jnp.where(m, ak, rbk)
    hv = jnp.where(m, av_, rbv)
    lk = jnp.where(m, rbk, ak)
    lv = jnp.where(m, rbv, av_)
    lk, lv = plsc.sort_key_val(lk, lv)
    hk, hv = plsc.sort_key_val(hk, hv)
    return lk, lv, hk, hv


def _top32_update(carry, ck, cv):
    tlk, tlv, thk, thv = carry
    rbk = lax.rev(tlk, (0,))
    rbv = lax.rev(tlv, (0,))
    m = ck >= rbk
    uk = jnp.where(m, ck, rbk)
    uv = jnp.where(m, cv, rbv)
    uk, uv = plsc.sort_key_val(uk, uv)
    nlk, nlv, nhk, nhv = _merge16(thk, thv, uk, uv)
    return nlk, nlv, nhk, nhv


def _attn_sc_body(alpha_hbm, hs_hbm, wsca_hbm, predq_hbm, out_hbm,
                  av_ref, ckf_ref, cvf_ref, st32k_ref, st32v_ref,
                  i32_ref, w32_ref, idx2_ref, rows_ref, wsca_ref,
                  pq_ref, st16_ref, p256_ref,
                  shk_ref, shv_ref, shw_ref, shi_ref, shp_ref, sem):
    wid = lax.axis_index("s")
    iota = lax.iota(jnp.int32, 16)
    ninf = jnp.float32(-jnp.inf)
    init = (jnp.full((16,), ninf), jnp.zeros((16,), jnp.int32),
            jnp.full((16,), ninf), jnp.zeros((16,), jnp.int32))

    pltpu.sync_copy(alpha_hbm.at[pl.ds(wid * _PER, _PER)], av_ref)
    pltpu.sync_copy(wsca_hbm, wsca_ref)

    base = wid * _PER

    def step(i, carry):
        c = av_ref[pl.ds(i * 16, 16)]
        ck, cv = plsc.sort_key_val(c, iota + (base + i * 16))
        return _top32_update(carry, ck, cv)

    tlk, tlv, thk, thv = lax.fori_loop(0, _CH, step, init)

    st32k_ref[pl.ds(0, 16)] = tlk
    st32k_ref[pl.ds(16, 16)] = thk
    st32v_ref[pl.ds(0, 16)] = tlv
    st32v_ref[pl.ds(16, 16)] = thv
    pltpu.sync_copy(st32k_ref, shk_ref.at[pl.ds(wid * 32, 32)])
    pltpu.sync_copy(st32v_ref, shv_ref.at[pl.ds(wid * 32, 32)])
    plsc.subcore_barrier()

    @pl.when(wid == 0)
    def _():
        pltpu.sync_copy(shk_ref, ckf_ref)
        pltpu.sync_copy(shv_ref, cvf_ref)

        def step2(i, carry):
            ck = ckf_ref[pl.ds(i * 16, 16)]
            cv = cvf_ref[pl.ds(i * 16, 16)]
            ck, cv = plsc.sort_key_val(ck, cv)
            return _top32_update(carry, ck, cv)

        tlk0, tlv0, thk0, thv0 = lax.fori_loop(0, 2 * _NS, step2, init)
        m0 = _splat_lane(st16_ref, thk0, 15)
        e_lo = jnp.exp(tlk0 - m0)
        e_hi = jnp.exp(thk0 - m0)
        zvec = _bfly_sum(st16_ref, e_lo + e_hi)
        w32_ref[pl.ds(0, 16)] = e_lo / zvec
        w32_ref[pl.ds(16, 16)] = e_hi / zvec
        i32_ref[pl.ds(0, 16)] = tlv0
        i32_ref[pl.ds(16, 16)] = thv0
        pltpu.sync_copy(w32_ref, shw_ref)
        pltpu.sync_copy(i32_ref, shi_ref)
    plsc.subcore_barrier()

    pltpu.sync_copy(shw_ref, w32_ref)
    pltpu.sync_copy(shi_ref, i32_ref)
    gidx = plsc.load_gather(i32_ref, [jnp.where(iota < 2, iota + 2 * wid, 0)])
    plsc.store_scatter(idx2_ref, [iota], gidx, mask=iota < 2)
    pltpu.async_copy(hs_hbm.at[idx2_ref], rows_ref, sem).wait()
    part = jnp.zeros((16,), jnp.float32)
    for r in range(2):
        acc = jnp.zeros((16,), jnp.float32)
        for c in range(HID // 16):
            acc = acc + rows_ref[r, 0, pl.ds(c * 16, 16)] * wsca_ref[pl.ds(c * 16, 16)]
        gw = plsc.load_gather(w32_ref, [jnp.zeros((16,), jnp.int32) + (2 * wid + r)])
        part = part + acc * gw
    st16_ref[...] = _bfly_sum(pq_ref, part)
    pltpu.sync_copy(st16_ref, shp_ref.at[pl.ds(wid * 16, 16)])
    plsc.subcore_barrier()

    @pl.when(wid == 0)
    def _():
        pltpu.sync_copy(shp_ref, p256_ref)
        pltpu.sync_copy(predq_hbm, pq_ref)
        tot = jnp.zeros((16,), jnp.float32)
        for r in range(_NS):
            tot = tot + p256_ref[pl.ds(r * 16, 16)]
        st16_ref[...] = tot + pq_ref[...]
        pltpu.sync_copy(st16_ref, out_hbm)


def _attn_sc(alpha, hs, wsca, predq16):
    mesh = plsc.VectorSubcoreMesh(core_axis_name="c", subcore_axis_name="s",
                                  num_cores=1)
    f32 = jnp.float32
    i32 = jnp.int32
    run = pl.kernel(
        _attn_sc_body,
        out_type=jax.ShapeDtypeStruct((16,), f32),
        mesh=mesh,
        compiler_params=pltpu.CompilerParams(needs_layout_passes=False),
        scratch_types=[
            pltpu.VMEM((_PER,), f32),
            pltpu.VMEM((32 * _NS,), f32),
            pltpu.VMEM((32 * _NS,), i32),
            pltpu.VMEM((32,), f32),
            pltpu.VMEM((32,), i32),
            pltpu.VMEM((32,), i32),
            pltpu.VMEM((32,), f32),
            pltpu.VMEM((2,), i32),
            pltpu.VMEM((2, 1, HID), f32),
            pltpu.VMEM((HID,), f32),
            pltpu.VMEM((16,), f32),
            pltpu.VMEM((16,), f32),
            pltpu.VMEM((16 * _NS,), f32),
            pltpu.VMEM_SHARED((32 * _NS,), f32),
            pltpu.VMEM_SHARED((32 * _NS,), i32),
            pltpu.VMEM_SHARED((32,), f32),
            pltpu.VMEM_SHARED((32,), i32),
            pltpu.VMEM_SHARED((16 * _NS,), f32),
            pltpu.SemaphoreType.DMA,
        ],
    )
    return run(alpha, hs, wsca, predq16)


def kernel(question, score, questions, hs, W_score, b_score, W_ih, W_hh, b_ih, b_hh):
    h0 = hs[T - 1, 0]
    alpha, h_new, predq = _stage_a(questions, question, score, h0, W_score,
                                   b_score, W_ih, W_hh, b_ih, b_hh)
    predq16 = jnp.broadcast_to(predq.reshape(1), (16,))
    pred16 = _attn_sc(alpha, hs, W_score[0, QUES:], predq16)
    pred = pred16[:1].reshape(1, 1)
    return pred, h_new.reshape(1, 1, HID)


# speedup vs baseline: 1.1051x; 1.0087x over previous
import functools

import jax
import jax.numpy as jnp
from jax import lax
from jax.experimental import pallas as pl
from jax.experimental.pallas import tpu as pltpu
from jax.experimental.pallas import tpu_sc as plsc

QUES = 2048
HID = 1024
T = 32768
K = 32

_INTERPRET = False

_GA = 32
_BT = T // _GA
_BH = HID // _GA


def _stage_a_body(sel_ref, qblk_ref, qrow_ref, h0r_ref, h0c_ref, wih_ref,
                  whh_ref, bih_ref, bhh_ref, wsc_ref, bsc_ref,
                  alpha_ref, h_ref, predq_ref):
    j = pl.program_id(0)
    q = qrow_ref[...]
    av = jax.lax.dot_general(
        q, qblk_ref[...], (((1,), (1,)), ((), ())),
        preferred_element_type=jnp.float32)
    alpha_ref[...] = av[0]

    gi3 = jax.lax.dot_general(
        wih_ref[...], q, (((2,), (1,)), ((), ())),
        preferred_element_type=jnp.float32)
    gh3 = jax.lax.dot_general(
        whh_ref[...], h0r_ref[...], (((2,), (1,)), ((), ())),
        preferred_element_type=jnp.float32)
    i_r, i_z, i_n = gi3[0] + bih_ref[0], gi3[1] + bih_ref[1], gi3[2] + bih_ref[2]
    h_r, h_z, h_n = gh3[0] + bhh_ref[0], gh3[1] + bhh_ref[1], gh3[2] + bhh_ref[2]
    r = jax.nn.sigmoid(i_r + h_r)
    z = jax.nn.sigmoid(i_z + h_z)
    n = jnp.tanh(i_n + r * h_n)
    h_ref[...] = (1.0 - z) * n + z * h0c_ref[...]

    @pl.when(j == 0)
    def _():
        predq_ref[0, 0] = jnp.sum(q * wsc_ref[:, :QUES]) + bsc_ref[0, 0]


def _stage_a(questions, question, score, h0, W_score, b_score,
             W_ih, W_hh, b_ih, b_hh):
    wih3 = W_ih.reshape(3, HID, 2 * QUES)
    whh3 = W_hh.reshape(3, HID, HID)
    bih3 = b_ih.reshape(3, HID, 1)
    bhh3 = b_hh.reshape(3, HID, 1)
    sel = (score[0] < 0.5).astype(jnp.int32).reshape(1)
    grid_spec = pltpu.PrefetchScalarGridSpec(
        num_scalar_prefetch=1,
        grid=(_GA,),
        in_specs=[
            pl.BlockSpec((_BT, QUES), lambda j, sel: (j, 0)),
            pl.BlockSpec((1, QUES), lambda j, sel: (0, 0)),
            pl.BlockSpec((1, HID), lambda j, sel: (0, 0)),
            pl.BlockSpec((_BH, 1), lambda j, sel: (j, 0)),
            pl.BlockSpec((3, _BH, QUES), lambda j, sel: (0, j, sel[0])),
            pl.BlockSpec((3, _BH, HID), lambda j, sel: (0, j, 0)),
            pl.BlockSpec((3, _BH, 1), lambda j, sel: (0, j, 0)),
            pl.BlockSpec((3, _BH, 1), lambda j, sel: (0, j, 0)),
            pl.BlockSpec((1, QUES + HID), lambda j, sel: (0, 0)),
            pl.BlockSpec((1, 1), lambda j, sel: (0, 0),
                         memory_space=pltpu.SMEM),
        ],
        out_specs=[
            pl.BlockSpec((_BT,), lambda j, sel: (j,)),
            pl.BlockSpec((_BH, 1), lambda j, sel: (j, 0)),
            pl.BlockSpec((1, 1), lambda j, sel: (0, 0),
                         memory_space=pltpu.SMEM),
        ],
    )
    return pl.pallas_call(
        _stage_a_body,
        grid_spec=grid_spec,
        out_shape=[
            jax.ShapeDtypeStruct((T,), jnp.float32),
            jax.ShapeDtypeStruct((HID, 1), jnp.float32),
            jax.ShapeDtypeStruct((1, 1), jnp.float32),
        ],
        interpret=_INTERPRET,
    )(sel, questions, question.reshape(1, QUES),
      h0.reshape(1, HID), h0.reshape(HID, 1), wih3, whh3, bih3, bhh3,
      W_score, b_score.reshape(1, 1))


def _attn_body(alpha_ref, hs_ref, wsca_ref, predq_ref, pred_ref,
               rows_ref, sems):
    a = alpha_ref[...]
    flat = jax.lax.broadcasted_iota(jnp.int32, a.shape, 0)
    neg = jnp.float32(-jnp.inf)
    big = jnp.int32(2**30)
    vals = []
    for t in range(K):
        m = jnp.max(a)
        eq = a == m
        fi = jnp.min(jnp.where(eq, flat, big))
        pltpu.make_async_copy(hs_ref.at[fi], rows_ref.at[t], sems.at[t]).start()
        vals.append(m)
        a = jnp.where(flat == fi, neg, a)
    m0 = vals[0]
    ws = [jnp.exp(v - m0) for v in vals]
    z = functools.reduce(lambda x, y: x + y, ws)
    acc = jnp.zeros((1, HID), dtype=jnp.float32)
    for t in range(K):
        pltpu.make_async_copy(hs_ref.at[0], rows_ref.at[t], sems.at[t]).wait()
        acc = acc + rows_ref[t] * (ws[t] / z)
    pred_ref[0, 0] = predq_ref[0, 0] + jnp.sum(acc * wsca_ref[...])


def _attn(alpha, hs, wsc_attn, predq):
    return pl.pallas_call(
        _attn_body,
        in_specs=[
            pl.BlockSpec(memory_space=pltpu.VMEM),
            pl.BlockSpec(memory_space=pl.ANY),
            pl.BlockSpec(memory_space=pltpu.VMEM),
            pl.BlockSpec(memory_space=pltpu.SMEM),
        ],
        out_specs=pl.BlockSpec(memory_space=pltpu.SMEM),
        out_shape=jax.ShapeDtypeStruct((1, 1), jnp.float32),
        scratch_shapes=[
            pltpu.VMEM((K, 1, HID), jnp.float32),
            pltpu.SemaphoreType.DMA((K,)),
        ],
        interpret=_INTERPRET,
    )(alpha, hs, wsc_attn, predq)


_NS = 16
_PER = T // _NS
_CH = _PER // 16


def _splat_lane(ref, x, lane):
    ref[...] = x
    return plsc.load_gather(ref, [jnp.zeros((16,), jnp.int32) + lane])


def _bfly_sum(ref, x):
    iota = lax.iota(jnp.int32, 16)
    v = x
    for sh in (8, 4, 2, 1):
        ref[...] = v
        v = v + plsc.load_gather(ref, [jnp.bitwise_xor(iota, sh)])
    return v


def _merge16(ak, av_, bk, bv):
    rbk = lax.rev(bk, (0,))
    rbv = lax.rev(bv, (0,))
    m = ak >= rbk
    hk = jnp.where(m, ak, rbk)
    hv = jnp.where(m, av_, rbv)
    lk = jnp.where(m, rbk, ak)
    lv = jnp.where(m, rbv, av_)
    lk, lv = plsc.sort_key_val(lk, lv)
    hk, hv = plsc.sort_key_val(hk, hv)
    return lk, lv, hk, hv


def _top32_update(carry, ck, cv):
    tlk, tlv, thk, thv = carry
    rbk = lax.rev(tlk, (0,))
    rbv = lax.rev(tlv, (0,))
    m = ck >= rbk
    uk = jnp.where(m, ck, rbk)
    uv = jnp.where(m, cv, rbv)
    uk, uv = plsc.sort_key_val(uk, uv)
    nlk, nlv, nhk, nhv = _merge16(thk, thv, uk, uv)
    return nlk, nlv, nhk, nhv


def _attn_sc_body(alpha_hbm, hs_hbm, wsca_hbm, predq_hbm, out_hbm,
                  av_ref, ckf_ref, cvf_ref, st32k_ref, st32v_ref,
                  i32_ref, w32_ref, idx2_ref, rows_ref, wsca_ref,
                  pq_ref, st16_ref, p256_ref,
                  shk_ref, shv_ref, shw_ref, shi_ref, shp_ref, sem):
    wid = lax.axis_index("s")
    iota = lax.iota(jnp.int32, 16)
    ninf = jnp.float32(-jnp.inf)
    init = (jnp.full((16,), ninf), jnp.zeros((16,), jnp.int32),
            jnp.full((16,), ninf), jnp.zeros((16,), jnp.int32))

    pltpu.sync_copy(alpha_hbm.at[pl.ds(wid * _PER, _PER)], av_ref)
    pltpu.sync_copy(wsca_hbm, wsca_ref)

    base = wid * _PER

    def step(i, carry):
        cA = av_ref[pl.ds(i * 32, 16)]
        cB = av_ref[pl.ds(i * 32 + 16, 16)]
        kA, vA = plsc.sort_key_val(cA, iota + (base + i * 32))
        kB, vB = plsc.sort_key_val(cB, iota + (base + i * 32 + 16))
        nA = _top32_update(carry[:4], kA, vA)
        nB = _top32_update(carry[4:], kB, vB)
        return nA + nB

    dual = lax.fori_loop(0, _CH // 2, step, init + init)
    carry = _top32_update(dual[:4], dual[4], dual[5])
    tlk, tlv, thk, thv = _top32_update(carry, dual[6], dual[7])

    st32k_ref[pl.ds(0, 16)] = tlk
    st32k_ref[pl.ds(16, 16)] = thk
    st32v_ref[pl.ds(0, 16)] = tlv
    st32v_ref[pl.ds(16, 16)] = thv
    pltpu.sync_copy(st32k_ref, shk_ref.at[pl.ds(wid * 32, 32)])
    pltpu.sync_copy(st32v_ref, shv_ref.at[pl.ds(wid * 32, 32)])
    plsc.subcore_barrier()

    @pl.when(wid == 0)
    def _():
        pltpu.sync_copy(shk_ref, ckf_ref)
        pltpu.sync_copy(shv_ref, cvf_ref)

        def step2(i, carry):
            cA = ckf_ref[pl.ds(i * 32, 16)]
            vA = cvf_ref[pl.ds(i * 32, 16)]
            cB = ckf_ref[pl.ds(i * 32 + 16, 16)]
            vB = cvf_ref[pl.ds(i * 32 + 16, 16)]
            kA, vA = plsc.sort_key_val(cA, vA)
            kB, vB = plsc.sort_key_val(cB, vB)
            nA = _top32_update(carry[:4], kA, vA)
            nB = _top32_update(carry[4:], kB, vB)
            return nA + nB

        dual2 = lax.fori_loop(0, _NS, step2, init + init)
        c2 = _top32_update(dual2[:4], dual2[4], dual2[5])
        tlk0, tlv0, thk0, thv0 = _top32_update(c2, dual2[6], dual2[7])
        m0 = _splat_lane(st16_ref, thk0, 15)
        e_lo = jnp.exp(tlk0 - m0)
        e_hi = jnp.exp(thk0 - m0)
        zvec = _bfly_sum(st16_ref, e_lo + e_hi)
        w32_ref[pl.ds(0, 16)] = e_lo / zvec
        w32_ref[pl.ds(16, 16)] = e_hi / zvec
        i32_ref[pl.ds(0, 16)] = tlv0
        i32_ref[pl.ds(16, 16)] = thv0
        pltpu.sync_copy(w32_ref, shw_ref)
        pltpu.sync_copy(i32_ref, shi_ref)
    plsc.subcore_barrier()

    pltpu.sync_copy(shw_ref, w32_ref)
    pltpu.sync_copy(shi_ref, i32_ref)
    gidx = plsc.load_gather(i32_ref, [jnp.where(iota < 2, iota + 2 * wid, 0)])
    plsc.store_scatter(idx2_ref, [iota], gidx, mask=iota < 2)
    pltpu.async_copy(hs_hbm.at[idx2_ref], rows_ref, sem).wait()
    part = jnp.zeros((16,), jnp.float32)
    for r in range(2):
        acc = jnp.zeros((16,), jnp.float32)
        for c in range(HID // 16):
            acc = acc + rows_ref[r, 0, pl.ds(c * 16, 16)] * wsca_ref[pl.ds(c * 16, 16)]
        gw = plsc.load_gather(w32_ref, [jnp.zeros((16,), jnp.int32) + (2 * wid + r)])
        part = part + acc * gw
    st16_ref[...] = _bfly_sum(pq_ref, part)
    pltpu.sync_copy(st16_ref, shp_ref.at[pl.ds(wid * 16, 16)])
    plsc.subcore_barrier()

    @pl.when(wid == 0)
    def _():
        pltpu.sync_copy(shp_ref, p256_ref)
        pltpu.sync_copy(predq_hbm, pq_ref)
        tot = jnp.zeros((16,), jnp.float32)
        for r in range(_NS):
            tot = tot + p256_ref[pl.ds(r * 16, 16)]
        st16_ref[...] = tot + pq_ref[...]
        pltpu.sync_copy(st16_ref, out_hbm)


def _attn_sc(alpha, hs, wsca, predq16):
    mesh = plsc.VectorSubcoreMesh(core_axis_name="c", subcore_axis_name="s",
                                  num_cores=1)
    f32 = jnp.float32
    i32 = jnp.int32
    run = pl.kernel(
        _attn_sc_body,
        out_type=jax.ShapeDtypeStruct((16,), f32),
        mesh=mesh,
        compiler_params=pltpu.CompilerParams(needs_layout_passes=False),
        scratch_types=[
            pltpu.VMEM((_PER,), f32),
            pltpu.VMEM((32 * _NS,), f32),
            pltpu.VMEM((32 * _NS,), i32),
            pltpu.VMEM((32,), f32),
            pltpu.VMEM((32,), i32),
            pltpu.VMEM((32,), i32),
            pltpu.VMEM((32,), f32),
            pltpu.VMEM((2,), i32),
            pltpu.VMEM((2, 1, HID), f32),
            pltpu.VMEM((HID,), f32),
            pltpu.VMEM((16,), f32),
            pltpu.VMEM((16,), f32),
            pltpu.VMEM((16 * _NS,), f32),
            pltpu.VMEM_SHARED((32 * _NS,), f32),
            pltpu.VMEM_SHARED((32 * _NS,), i32),
            pltpu.VMEM_SHARED((32,), f32),
            pltpu.VMEM_SHARED((32,), i32),
            pltpu.VMEM_SHARED((16 * _NS,), f32),
            pltpu.SemaphoreType.DMA,
        ],
    )
    return run(alpha, hs, wsca, predq16)


def kernel(question, score, questions, hs, W_score, b_score, W_ih, W_hh, b_ih, b_hh):
    h0 = hs[T - 1, 0]
    alpha, h_new, predq = _stage_a(questions, question, score, h0, W_score,
                                   b_score, W_ih, W_hh, b_ih, b_hh)
    predq16 = jnp.broadcast_to(predq.reshape(1), (16,))
    pred16 = _attn_sc(alpha, hs, W_score[0, QUES:], predq16)
    pred = pred16[:1].reshape(1, 1)
    return pred, h_new.reshape(1, 1, HID)
